# baseline (speedup 1.0000x reference)
import functools

import jax
import jax.numpy as jnp
from jax import lax
from jax.experimental import pallas as pl
from jax.experimental.pallas import tpu as pltpu
from jax.experimental.pallas import tpu_sc as plsc

_N = 10000
_E = 320000
_D = 128

_NW = 32
_SLICES = (38400, 89600, 89600, 89600, 12800)
_S = len(_SLICES)
_STARTS = tuple(sum(_SLICES[:i]) for i in range(_S))
_EPW = _E // _NW
_CH = 80
_NCH = _EPW // _CH
_NPAD = 10240
_RPT = _NPAD // 16

_BN = 1000
_BE = 2560


def _lrelu(h):
  return jnp.where(h >= 0, h, 0.01 * h)


def _dot(a, b):
  return lax.dot_general(a, b, (((1,), (0,)), ((), ())),
                         preferred_element_type=jnp.float32)


def _dot_t(a, b):
  return lax.dot_general(a, b, (((0,), (0,)), ((), ())),
                         preferred_element_type=jnp.float32)


def _sc_mesh():
  return plsc.VectorSubcoreMesh(core_axis_name="c", subcore_axis_name="s")


def _sc_gather_add(table_a, table_b, src, dst, s_idx):
  size = _SLICES[s_idx]
  epws = size // _NW
  nchs = epws // _CH

  @functools.partial(
      pl.kernel,
      out_type=jax.ShapeDtypeStruct((size, _D), jnp.float32),
      mesh=_sc_mesh(),
      scratch_types=[
          pltpu.VMEM((epws,), jnp.int32),
          pltpu.VMEM((epws,), jnp.int32),
          [pltpu.VMEM((_CH, _D), jnp.float32)] * 2,
          [pltpu.VMEM((_CH, _D), jnp.float32)] * 2,
          [pltpu.SemaphoreType.DMA] * 2,
          [pltpu.SemaphoreType.DMA] * 2,
          [pltpu.SemaphoreType.DMA] * 2,
      ],
  )
  def k(a_hbm, b_hbm, src_hbm, dst_hbm, out_hbm,
        idx_s, idx_d, rows_a, rows_b, sem_a, sem_b, sem_w):
    wid = lax.axis_index("s") * 2 + lax.axis_index("c")
    local = wid * epws
    first = _STARTS[s_idx] + local

    pltpu.sync_copy(src_hbm.at[pl.ds(first, epws)], idx_s)
    pltpu.sync_copy(dst_hbm.at[pl.ds(first, epws)], idx_d)

    def start(slot, i):
      @pl.when(i >= 2)
      def _():
        pltpu.make_async_copy(
            rows_a[slot], out_hbm.at[pl.ds(local, _CH), :],
            sem_w[slot]).wait()

      sl = pl.ds(i * _CH, _CH)
      pltpu.async_copy(a_hbm.at[idx_s.at[sl]], rows_a[slot], sem_a[slot])
      pltpu.async_copy(b_hbm.at[idx_d.at[sl]], rows_b[slot], sem_b[slot])

    def finish(slot, i):
      sl = pl.ds(i * _CH, _CH)
      pltpu.make_async_copy(a_hbm.at[idx_s.at[sl]], rows_a[slot],
                            sem_a[slot]).wait()
      pltpu.make_async_copy(b_hbm.at[idx_d.at[sl]], rows_b[slot],
                            sem_b[slot]).wait()
      ra, rb = rows_a[slot], rows_b[slot]

      def add_row(r, carry):
        for c in range(_D // 16):
          sl = pl.ds(c * 16, 16)
          ra[r, sl] = ra[r, sl] + rb[r, sl]
        return carry

      lax.fori_loop(0, _CH, add_row, 0)
      pltpu.async_copy(ra, out_hbm.at[pl.ds(local + i * _CH, _CH), :],
                       sem_w[slot])

    start(0, 0)

    def body(g, carry):
      start(1, 2 * g + 1)
      finish(0, 2 * g)
      start(0, 2 * g + 2)
      finish(1, 2 * g + 1)
      return carry

    lax.fori_loop(0, (nchs - 1) // 2, body, 0)
    finish(0, nchs - 1)
    pltpu.make_async_copy(rows_a[0], out_hbm.at[pl.ds(local, _CH), :],
                          sem_w[0]).wait()
    pltpu.make_async_copy(rows_a[1], out_hbm.at[pl.ds(local, _CH), :],
                          sem_w[1]).wait()

  return k(table_a, table_b, src, dst)


def _sc_segsum(e, dst, zeros_init):
  @functools.partial(
      pl.kernel,
      out_type=jax.ShapeDtypeStruct((2, _NPAD, _D), jnp.float32),
      mesh=_sc_mesh(),
      scratch_types=[
          [pltpu.VMEM((_CH,), jnp.int32)] * 2,
          [pltpu.VMEM((_CH, _D), jnp.float32)] * 2,
          [pltpu.SemaphoreType.DMA] * 2,
          [pltpu.SemaphoreType.DMA] * 2,
          pltpu.VMEM_SHARED((_NPAD, _D), jnp.float32),
      ],
  )
  def k(e_hbm, dst_hbm, z_hbm, out_hbm, idx_d, rows, sem_i, sem_r, acc):
    c = lax.axis_index("c")
    s = lax.axis_index("s")
    wid = s * 2 + c
    first = wid * _EPW

    pltpu.sync_copy(z_hbm, acc.at[pl.ds(s * _RPT, _RPT), :])
    plsc.subcore_barrier()

    def start(slot, i):
      base = first + i * _CH
      pltpu.async_copy(dst_hbm.at[pl.ds(base, _CH)], idx_d[slot],
                       sem_i[slot])
      pltpu.async_copy(e_hbm.at[pl.ds(base, _CH), :], rows[slot],
                       sem_r[slot])

    def finish(slot, i):
      base = first + i * _CH
      pltpu.make_async_copy(dst_hbm.at[pl.ds(base, _CH)], idx_d[slot],
                            sem_i[slot]).wait()
      pltpu.make_async_copy(e_hbm.at[pl.ds(base, _CH), :], rows[slot],
                            sem_r[slot]).wait()
      pltpu.sync_copy(rows[slot], acc.at[idx_d[slot]], add=True)

    start(0, 0)

    def body(g, carry):
      start(1, 2 * g + 1)
      finish(0, 2 * g)
      start(0, 2 * g + 2)
      finish(1, 2 * g + 1)
      return carry

    lax.fori_loop(0, (_NCH - 1) // 2, body, 0)
    finish(0, _NCH - 1)
    plsc.subcore_barrier()

    pltpu.sync_copy(acc.at[pl.ds(s * _RPT, _RPT), :],
                    out_hbm.at[c, pl.ds(s * _RPT, _RPT), :])

  return k(e, dst, zeros_init)


def _wspec(shape):
  return pl.BlockSpec(shape, lambda i: (0,) * len(shape))


def _rspec(r):
  return pl.BlockSpec((_D, _D), lambda i, r=r: (r, 0))


def _enc_node_call(x, wen, ben, wa, wb):
  def body(x_ref, wen_ref, ben_ref, wa_ref, wb_ref, v_ref, a_ref, b_ref):
    v = _lrelu(_dot(x_ref[...], wen_ref[...]) + ben_ref[...])
    v_ref[...] = v
    a_ref[...] = _dot(v, wa_ref[...])
    b_ref[...] = _dot(v, wb_ref[...])

  rows = pl.BlockSpec((_BN, _D), lambda i: (i, 0))
  out = jax.ShapeDtypeStruct((_N, _D), jnp.float32)
  return pl.pallas_call(
      body,
      grid=(_N // _BN,),
      in_specs=[rows, _wspec((_D, _D)), _wspec((1, _D)),
                _rspec(0), _rspec(1)],
      out_specs=[rows, rows, rows],
      out_shape=[out, out, out],
  )(x, wen, ben, wa, wb)


def _edge_call(first, s_idx, e_in, gab, alias_buf, weights):
  nb = _SLICES[s_idx] // _BE
  b0 = _STARTS[s_idx] // _BE
  rows_sl = pl.BlockSpec((_BE, _D), lambda i: (i, 0))
  rows_full = pl.BlockSpec((_BE, _D), lambda i: (i + b0, 0))
  acc = pl.BlockSpec((1, _D), lambda i: (0, 0))
  any_spec = pl.BlockSpec(memory_space=pl.ANY)
  with_alias = alias_buf is not None

  if first:
    e_spec = pl.BlockSpec((4, _BE), lambda i: (0, i + b0))
    wee, bee, w1c, ce, w2, b2 = weights

    def body(e_ref, gab_ref, wee_ref, bee_ref, w1c_ref, ce_ref,
             w2_ref, b2_ref, e_out_ref, se_ref):
      e0 = _lrelu(_dot_t(e_ref[...], wee_ref[...]) + bee_ref[...])
      h = _lrelu(_dot(e0, w1c_ref[...]) + gab_ref[...] + ce_ref[...])
      en = e0 + _dot(h, w2_ref[...]) + b2_ref[...]
      e_out_ref[...] = en

      @pl.when(pl.program_id(0) == 0)
      def _():
        se_ref[...] = jnp.zeros_like(se_ref)

      se_ref[...] += jnp.sum(en, axis=0, keepdims=True)

    w_specs = [_wspec((4, _D)), _wspec((1, _D)), _rspec(2),
               _wspec((1, _D)), _wspec((_D, _D)), _wspec((1, _D))]
  else:
    e_spec = rows_full
    w1c, ce, w2, b2 = weights

    def body(e_ref, gab_ref, w1c_ref, ce_ref, w2_ref, b2_ref,
             e_out_ref, se_ref):
      e0 = e_ref[...]
      h = _lrelu(_dot(e0, w1c_ref[...]) + gab_ref[...] + ce_ref[...])
      en = e0 + _dot(h, w2_ref[...]) + b2_ref[...]
      e_out_ref[...] = en

      @pl.when(pl.program_id(0) == 0)
      def _():
        se_ref[...] = jnp.zeros_like(se_ref)

      se_ref[...] += jnp.sum(en, axis=0, keepdims=True)

    w_specs = [_rspec(2), _wspec((1, _D)), _wspec((_D, _D)),
               _wspec((1, _D))]

  if with_alias:
    outer_body = lambda *refs: body(*refs[1:])
    in_specs = [any_spec, e_spec, rows_sl] + w_specs
    args = (alias_buf, e_in, gab, *weights)
    io_alias = {0: 0}
  else:
    outer_body = body
    in_specs = [e_spec, rows_sl] + w_specs
    args = (e_in, gab, *weights)
    io_alias = {}

  return pl.pallas_call(
      outer_body,
      grid=(nb,),
      in_specs=in_specs,
      out_specs=[rows_full, acc],
      out_shape=[jax.ShapeDtypeStruct((_E, _D), jnp.float32),
                 jax.ShapeDtypeStruct((1, _D), jnp.float32)],
      input_output_aliases=io_alias,
  )(*args)


def _node_call(last, v, aggp, weights):
  rows = pl.BlockSpec((_BN, _D), lambda i: (i, 0))
  pp = pl.BlockSpec((2, _BN, _D), lambda i: (0, i, 0))
  acc = pl.BlockSpec((1, _D), lambda i: (0, 0))
  outn = jax.ShapeDtypeStruct((_N, _D), jnp.float32)
  out1 = jax.ShapeDtypeStruct((1, _D), jnp.float32)

  if not last:
    cn, wn1a, wn1b, wn2, bn2, wa_n, wb_n = weights

    def body(v_ref, pp_ref, cn_ref, wn1a_ref, wn1b_ref, wn2_ref,
             bn2_ref, wa_ref, wb_ref, vout_ref, sv_ref, a_ref, b_ref):
      agg = pp_ref[0] + pp_ref[1]
      h = _lrelu(_dot(v_ref[...], wn1a_ref[...]) + _dot(agg, wn1b_ref[...])
                 + cn_ref[...])
      vn = v_ref[...] + _dot(h, wn2_ref[...]) + bn2_ref[...]
      vout_ref[...] = vn

      @pl.when(pl.program_id(0) == 0)
      def _():
        sv_ref[...] = jnp.zeros_like(sv_ref)

      sv_ref[...] += jnp.sum(vn, axis=0, keepdims=True)
      a_ref[...] = _dot(vn, wa_ref[...])
      b_ref[...] = _dot(vn, wb_ref[...])

    in_specs = [rows, pp, _wspec((1, _D)), _rspec(0),
                _rspec(1), _wspec((_D, _D)), _wspec((1, _D)),
                _rspec(0), _rspec(1)]
    out_specs = [rows, acc, rows, rows]
    out_shape = [outn, out1, outn, outn]
  else:
    cn, wn1a, wn1b, wn2, bn2, wd1, bd1, wd2p, bd2p = weights

    def body(v_ref, pp_ref, cn_ref, wn1a_ref, wn1b_ref, wn2_ref,
             bn2_ref, wd1_ref, bd1_ref, wd2_ref, bd2_ref, o_ref, sv_ref):
      agg = pp_ref[0] + pp_ref[1]
      h = _lrelu(_dot(v_ref[...], wn1a_ref[...]) + _dot(agg, wn1b_ref[...])
                 + cn_ref[...])
      vn = v_ref[...] + _dot(h, wn2_ref[...]) + bn2_ref[...]

      @pl.when(pl.program_id(0) == 0)
      def _():
        sv_ref[...] = jnp.zeros_like(sv_ref)

      sv_ref[...] += jnp.sum(vn, axis=0, keepdims=True)
      h2 = _lrelu(_dot(vn, wd1_ref[...]) + bd1_ref[...])
      o_ref[...] = _dot(h2, wd2_ref[...]) + bd2_ref[...]

    in_specs = [rows, pp, _wspec((1, _D)), _rspec(0),
                _rspec(1), _wspec((_D, _D)), _wspec((1, _D)),
                _wspec((_D, _D)), _wspec((1, _D)), _wspec((_D, _D)),
                _wspec((1, _D))]
    out_specs = [rows, acc]
    out_shape = [outn, out1]

  return pl.pallas_call(
      body,
      grid=(_N // _BN,),
      in_specs=in_specs,
      out_specs=out_specs,
      out_shape=out_shape,
  )(v, aggp, *weights)


def _g_init_call(u, weg, beg, ew1_full, b1e, nw1_full, bn1):
  def body(u_ref, weg_ref, beg_ref, ew1_ref, b1e_ref, nw1_ref, bn1_ref,
           g_ref, ce_ref, cn_ref):
    g = _lrelu(_dot(u_ref[...], weg_ref[...]) + beg_ref[...])
    g_ref[...] = g
    ce_ref[...] = _dot(g, ew1_ref[...]) + b1e_ref[...]
    cn_ref[...] = _dot(g, nw1_ref[...]) + bn1_ref[...]

  out1 = jax.ShapeDtypeStruct((1, _D), jnp.float32)
  specs = [_wspec((1, 8)), _wspec((8, _D)), _wspec((1, _D)),
           _rspec(3), _wspec((1, _D)), _rspec(2), _wspec((1, _D))]
  return pl.pallas_call(body, grid=(1,), in_specs=specs,
                        out_specs=[_wspec((1, _D))] * 3,
                        out_shape=[out1, out1, out1])(
      u, weg, beg, ew1_full, b1e, nw1_full, bn1)


def _g_update_call(g, sv, se, gw, nxt):
  gw1_full, bg1, wg2, bg2 = gw
  out1 = jax.ShapeDtypeStruct((1, _D), jnp.float32)
  gspecs = [_wspec((1, _D)), _wspec((1, _D)), _wspec((_S, _D)),
            _rspec(0), _rspec(1), _rspec(2), _wspec((1, _D)),
            _wspec((_D, _D)), _wspec((1, _D))]

  if nxt is None:
    def body(g_ref, sv_ref, se_ref, a_ref, b_ref, c_ref, bg1_ref, w2_ref,
             bg2_ref, gout_ref):
      mv = sv_ref[...] * (1.0 / _N)
      me = jnp.sum(se_ref[...], axis=0, keepdims=True) * (1.0 / _E)
      h = _lrelu(_dot(g_ref[...], a_ref[...]) + _dot(mv, b_ref[...])
                 + _dot(me, c_ref[...]) + bg1_ref[...])
      gout_ref[...] = g_ref[...] + _dot(h, w2_ref[...]) + bg2_ref[...]

    return pl.pallas_call(body, grid=(1,), in_specs=gspecs,
                          out_specs=[_wspec((1, _D))], out_shape=[out1])(
        g, sv, se, gw1_full, gw1_full, gw1_full, bg1, wg2, bg2)[0]

  ew1_n, b1e_n, nw1_n, bn1_n = nxt

  def body(g_ref, sv_ref, se_ref, a_ref, b_ref, c_ref, bg1_ref, w2_ref,
           bg2_ref, w1d_ref, b1e_ref, wn1c_ref, bn1_ref,
           gout_ref, ce_ref, cn_ref):
    mv = sv_ref[...] * (1.0 / _N)
    me = jnp.sum(se_ref[...], axis=0, keepdims=True) * (1.0 / _E)
    h = _lrelu(_dot(g_ref[...], a_ref[...]) + _dot(mv, b_ref[...])
               + _dot(me, c_ref[...]) + bg1_ref[...])
    gn = g_ref[...] + _dot(h, w2_ref[...]) + bg2_ref[...]
    gout_ref[...] = gn
    ce_ref[...] = _dot(gn, w1d_ref[...]) + b1e_ref[...]
    cn_ref[...] = _dot(gn, wn1c_ref[...]) + bn1_ref[...]

  specs = gspecs + [_rspec(3), _wspec((1, _D)), _rspec(2), _wspec((1, _D))]
  return pl.pallas_call(body, grid=(1,), in_specs=specs,
                        out_specs=[_wspec((1, _D))] * 3,
                        out_shape=[out1, out1, out1])(
      g, sv, se, gw1_full, gw1_full, gw1_full, bg1, wg2, bg2,
      ew1_n, b1e_n, nw1_n, bn1_n)


def _row(b):
  return b.reshape(1, -1)


def kernel(x, edge_attr, u, edge_index, params):
  p = params
  src = edge_index[0]
  dst = edge_index[1]
  ea_t = edge_attr.T
  wee = p["enc_edge"]["W"]
  bee = _row(p["enc_edge"]["b"])

  blk = p["blocks"]
  ew1 = [b["edge"]["l1"]["W"] for b in blk]
  b1e = [_row(b["edge"]["l1"]["b"]) for b in blk]
  ew2 = [b["edge"]["l2"]["W"] for b in blk]
  b2e = [_row(b["edge"]["l2"]["b"]) for b in blk]

  nw1 = [b["node"]["l1"]["W"] for b in blk]
  bn1 = [_row(b["node"]["l1"]["b"]) for b in blk]
  nw2 = [b["node"]["l2"]["W"] for b in blk]
  bn2 = [_row(b["node"]["l2"]["b"]) for b in blk]

  gws = [(b["glob"]["l1"]["W"], _row(b["glob"]["l1"]["b"]),
          b["glob"]["l2"]["W"], _row(b["glob"]["l2"]["b"])) for b in blk]

  wd1 = p["dec1"]["W"]
  bd1 = _row(p["dec1"]["b"])
  wd2p = jnp.pad(p["dec2"]["W"], ((0, 0), (0, _D - 3)))
  bd2p = jnp.pad(_row(p["dec2"]["b"]), ((0, 0), (0, _D - 3)))

  zeros_init = jnp.zeros((_RPT, _D), jnp.float32)

  v0, a1, b1 = _enc_node_call(x, p["enc_node"]["W"], _row(p["enc_node"]["b"]),
                              ew1[0], ew1[0])
  g0, ce1, cn1 = _g_init_call(u, p["enc_glob"]["W"], _row(p["enc_glob"]["b"]),
                              ew1[0], b1e[0], nw1[0], bn1[0])

  e1 = None
  ses = []
  for si in range(_S):
    gab = _sc_gather_add(a1, b1, src, dst, si)
    e1, se = _edge_call(True, si, ea_t, gab, e1,
                        (wee, bee, ew1[0], ce1, ew2[0], b2e[0]))
    ses.append(se)
  se1 = jnp.concatenate(ses, axis=0)
  agg1 = _sc_segsum(e1, dst, zeros_init)
  v1, sv1, a2, b2 = _node_call(False, v0, agg1,
                               (cn1, nw1[0], nw1[0], nw2[0], bn2[0],
                                ew1[1], ew1[1]))
  g1, ce2, cn2 = _g_update_call(g0, sv1, se1, gws[0],
                                (ew1[1], b1e[1], nw1[1], bn1[1]))

  e2 = None
  ses = []
  for si in range(_S):
    gab = _sc_gather_add(a2, b2, src, dst, si)
    e2, se = _edge_call(False, si, e1, gab, e2,
                        (ew1[1], ce2, ew2[1], b2e[1]))
    ses.append(se)
  se2 = jnp.concatenate(ses, axis=0)
  agg2 = _sc_segsum(e2, dst, zeros_init)
  opad, sv2 = _node_call(True, v1, agg2,
                         (cn2, nw1[1], nw1[1], nw2[1], bn2[1],
                          wd1, bd1, wd2p, bd2p))
  g2 = _g_update_call(g1, sv2, se2, gws[1], None)

  node_out = opad[:, :3]
  return (node_out, e2, g2)


# speedup vs baseline: 1.0075x; 1.0075x over previous
import functools

import jax
import jax.numpy as jnp
from jax import lax
from jax.experimental import pallas as pl
from jax.experimental.pallas import tpu as pltpu
from jax.experimental.pallas import tpu_sc as plsc

_N = 10000
_E = 320000
_D = 128

_NW = 32
_S = 5
_ES = _E // _S
_EPW = _E // _NW
_EPWS = _ES // _NW
_CH = 80
_NCH = _EPW // _CH
_NCHS = _EPWS // _CH
_NPAD = 10240
_RPT = _NPAD // 16

_BN = 1000
_BE = 2560


def _lrelu(h):
  return jnp.where(h >= 0, h, 0.01 * h)


def _dot(a, b):
  return lax.dot_general(a, b, (((1,), (0,)), ((), ())),
                         preferred_element_type=jnp.float32)


def _dot_t(a, b):
  return lax.dot_general(a, b, (((0,), (0,)), ((), ())),
                         preferred_element_type=jnp.float32)


def _sc_mesh():
  return plsc.VectorSubcoreMesh(core_axis_name="c", subcore_axis_name="s")


def _sc_gather_add(table_a, table_b, src, dst, s_idx):
  @functools.partial(
      pl.kernel,
      out_type=jax.ShapeDtypeStruct((_ES, _D), jnp.float32),
      mesh=_sc_mesh(),
      scratch_types=[
          pltpu.VMEM((_EPWS,), jnp.int32),
          pltpu.VMEM((_EPWS,), jnp.int32),
          [pltpu.VMEM((_CH, _D), jnp.float32)] * 2,
          [pltpu.VMEM((_CH, _D), jnp.float32)] * 2,
          [pltpu.SemaphoreType.DMA] * 2,
          [pltpu.SemaphoreType.DMA] * 2,
          [pltpu.SemaphoreType.DMA] * 2,
      ],
  )
  def k(a_hbm, b_hbm, src_hbm, dst_hbm, out_hbm,
        idx_s, idx_d, rows_a, rows_b, sem_a, sem_b, sem_w):
    wid = lax.axis_index("s") * 2 + lax.axis_index("c")
    local = wid * _EPWS
    first = s_idx * _ES + local

    pltpu.sync_copy(src_hbm.at[pl.ds(first, _EPWS)], idx_s)
    pltpu.sync_copy(dst_hbm.at[pl.ds(first, _EPWS)], idx_d)

    def start(slot, i):
      @pl.when(i >= 2)
      def _():
        pltpu.make_async_copy(
            rows_a[slot], out_hbm.at[pl.ds(local, _CH), :],
            sem_w[slot]).wait()

      sl = pl.ds(i * _CH, _CH)
      pltpu.async_copy(a_hbm.at[idx_s.at[sl]], rows_a[slot], sem_a[slot])
      pltpu.async_copy(b_hbm.at[idx_d.at[sl]], rows_b[slot], sem_b[slot])

    def finish(slot, i):
      sl = pl.ds(i * _CH, _CH)
      pltpu.make_async_copy(a_hbm.at[idx_s.at[sl]], rows_a[slot],
                            sem_a[slot]).wait()
      pltpu.make_async_copy(b_hbm.at[idx_d.at[sl]], rows_b[slot],
                            sem_b[slot]).wait()
      ra, rb = rows_a[slot], rows_b[slot]

      def add_row(r, carry):
        for c in range(_D // 16):
          sl = pl.ds(c * 16, 16)
          ra[r, sl] = ra[r, sl] + rb[r, sl]
        return carry

      lax.fori_loop(0, _CH, add_row, 0)
      pltpu.async_copy(ra, out_hbm.at[pl.ds(local + i * _CH, _CH), :],
                       sem_w[slot])

    start(0, 0)

    def body(g, carry):
      start(1, 2 * g + 1)
      finish(0, 2 * g)
      start(0, 2 * g + 2)
      finish(1, 2 * g + 1)
      return carry

    lax.fori_loop(0, (_NCHS - 1) // 2, body, 0)
    finish(0, _NCHS - 1)
    pltpu.make_async_copy(rows_a[0], out_hbm.at[pl.ds(local, _CH), :],
                          sem_w[0]).wait()
    pltpu.make_async_copy(rows_a[1], out_hbm.at[pl.ds(local, _CH), :],
                          sem_w[1]).wait()

  return k(table_a, table_b, src, dst)


def _sc_segsum(e, dst, zeros_init):
  @functools.partial(
      pl.kernel,
      out_type=jax.ShapeDtypeStruct((2, _NPAD, _D), jnp.float32),
      mesh=_sc_mesh(),
      scratch_types=[
          [pltpu.VMEM((_CH,), jnp.int32)] * 2,
          [pltpu.VMEM((_CH, _D), jnp.float32)] * 2,
          [pltpu.SemaphoreType.DMA] * 2,
          [pltpu.SemaphoreType.DMA] * 2,
          pltpu.VMEM_SHARED((_NPAD, _D), jnp.float32),
      ],
  )
  def k(e_hbm, dst_hbm, z_hbm, out_hbm, idx_d, rows, sem_i, sem_r, acc):
    c = lax.axis_index("c")
    s = lax.axis_index("s")
    wid = s * 2 + c
    first = wid * _EPW

    pltpu.sync_copy(z_hbm, acc.at[pl.ds(s * _RPT, _RPT), :])
    plsc.subcore_barrier()

    def start(slot, i):
      base = first + i * _CH
      pltpu.async_copy(dst_hbm.at[pl.ds(base, _CH)], idx_d[slot],
                       sem_i[slot])
      pltpu.async_copy(e_hbm.at[pl.ds(base, _CH), :], rows[slot],
                       sem_r[slot])

    def finish(slot, i):
      base = first + i * _CH
      pltpu.make_async_copy(dst_hbm.at[pl.ds(base, _CH)], idx_d[slot],
                            sem_i[slot]).wait()
      pltpu.make_async_copy(e_hbm.at[pl.ds(base, _CH), :], rows[slot],
                            sem_r[slot]).wait()
      pltpu.sync_copy(rows[slot], acc.at[idx_d[slot]], add=True)

    start(0, 0)

    def body(g, carry):
      start(1, 2 * g + 1)
      finish(0, 2 * g)
      start(0, 2 * g + 2)
      finish(1, 2 * g + 1)
      return carry

    lax.fori_loop(0, (_NCH - 1) // 2, body, 0)
    finish(0, _NCH - 1)
    plsc.subcore_barrier()

    pltpu.sync_copy(acc.at[pl.ds(s * _RPT, _RPT), :],
                    out_hbm.at[c, pl.ds(s * _RPT, _RPT), :])

  return k(e, dst, zeros_init)


def _wspec(shape):
  return pl.BlockSpec(shape, lambda i: (0,) * len(shape))


def _rspec(r):
  return pl.BlockSpec((_D, _D), lambda i, r=r: (r, 0))


def _enc_node_call(x, wen, ben, wa, wb):
  def body(x_ref, wen_ref, ben_ref, wa_ref, wb_ref, v_ref, a_ref, b_ref):
    v = _lrelu(_dot(x_ref[...], wen_ref[...]) + ben_ref[...])
    v_ref[...] = v
    a_ref[...] = _dot(v, wa_ref[...])
    b_ref[...] = _dot(v, wb_ref[...])

  rows = pl.BlockSpec((_BN, _D), lambda i: (i, 0))
  out = jax.ShapeDtypeStruct((_N, _D), jnp.float32)
  return pl.pallas_call(
      body,
      grid=(_N // _BN,),
      in_specs=[rows, _wspec((_D, _D)), _wspec((1, _D)),
                _rspec(0), _rspec(1)],
      out_specs=[rows, rows, rows],
      out_shape=[out, out, out],
  )(x, wen, ben, wa, wb)


def _edge_call(first, s_idx, e_in, gab, alias_buf, weights):
  nb = _ES // _BE
  rows_sl = pl.BlockSpec((_BE, _D), lambda i: (i, 0))
  rows_full = pl.BlockSpec((_BE, _D), lambda i: (i + s_idx * nb, 0))
  acc = pl.BlockSpec((1, _D), lambda i: (0, 0))
  any_spec = pl.BlockSpec(memory_space=pl.ANY)
  with_alias = alias_buf is not None

  if first:
    e_spec = pl.BlockSpec((4, _BE), lambda i: (0, i + s_idx * nb))
    wee, bee, w1c, ce, w2, b2 = weights

    def body(e_ref, gab_ref, wee_ref, bee_ref, w1c_ref, ce_ref,
             w2_ref, b2_ref, e_out_ref, se_ref):
      e0 = _lrelu(_dot_t(e_ref[...], wee_ref[...]) + bee_ref[...])
      h = _lrelu(_dot(e0, w1c_ref[...]) + gab_ref[...] + ce_ref[...])
      en = e0 + _dot(h, w2_ref[...]) + b2_ref[...]
      e_out_ref[...] = en

      @pl.when(pl.program_id(0) == 0)
      def _():
        se_ref[...] = jnp.zeros_like(se_ref)

      se_ref[...] += jnp.sum(en, axis=0, keepdims=True)

    w_specs = [_wspec((4, _D)), _wspec((1, _D)), _rspec(2),
               _wspec((1, _D)), _wspec((_D, _D)), _wspec((1, _D))]
  else:
    e_spec = rows_full
    w1c, ce, w2, b2 = weights

    def body(e_ref, gab_ref, w1c_ref, ce_ref, w2_ref, b2_ref,
             e_out_ref, se_ref):
      e0 = e_ref[...]
      h = _lrelu(_dot(e0, w1c_ref[...]) + gab_ref[...] + ce_ref[...])
      en = e0 + _dot(h, w2_ref[...]) + b2_ref[...]
      e_out_ref[...] = en

      @pl.when(pl.program_id(0) == 0)
      def _():
        se_ref[...] = jnp.zeros_like(se_ref)

      se_ref[...] += jnp.sum(en, axis=0, keepdims=True)

    w_specs = [_rspec(2), _wspec((1, _D)), _wspec((_D, _D)),
               _wspec((1, _D))]

  if with_alias:
    outer_body = lambda *refs: body(*refs[1:])
    in_specs = [any_spec, e_spec, rows_sl] + w_specs
    args = (alias_buf, e_in, gab, *weights)
    io_alias = {0: 0}
  else:
    outer_body = body
    in_specs = [e_spec, rows_sl] + w_specs
    args = (e_in, gab, *weights)
    io_alias = {}

  return pl.pallas_call(
      outer_body,
      grid=(nb,),
      in_specs=in_specs,
      out_specs=[rows_full, acc],
      out_shape=[jax.ShapeDtypeStruct((_E, _D), jnp.float32),
                 jax.ShapeDtypeStruct((1, _D), jnp.float32)],
      input_output_aliases=io_alias,
  )(*args)


def _node_call(last, v, aggp, weights):
  rows = pl.BlockSpec((_BN, _D), lambda i: (i, 0))
  pp = pl.BlockSpec((2, _BN, _D), lambda i: (0, i, 0))
  acc = pl.BlockSpec((1, _D), lambda i: (0, 0))
  outn = jax.ShapeDtypeStruct((_N, _D), jnp.float32)
  out1 = jax.ShapeDtypeStruct((1, _D), jnp.float32)

  if not last:
    cn, wn1a, wn1b, wn2, bn2, wa_n, wb_n = weights

    def body(v_ref, pp_ref, cn_ref, wn1a_ref, wn1b_ref, wn2_ref,
             bn2_ref, wa_ref, wb_ref, vout_ref, sv_ref, a_ref, b_ref):
      agg = pp_ref[0] + pp_ref[1]
      h = _lrelu(_dot(v_ref[...], wn1a_ref[...]) + _dot(agg, wn1b_ref[...])
                 + cn_ref[...])
      vn = v_ref[...] + _dot(h, wn2_ref[...]) + bn2_ref[...]
      vout_ref[...] = vn

      @pl.when(pl.program_id(0) == 0)
      def _():
        sv_ref[...] = jnp.zeros_like(sv_ref)

      sv_ref[...] += jnp.sum(vn, axis=0, keepdims=True)
      a_ref[...] = _dot(vn, wa_ref[...])
      b_ref[...] = _dot(vn, wb_ref[...])

    in_specs = [rows, pp, _wspec((1, _D)), _rspec(0),
                _rspec(1), _wspec((_D, _D)), _wspec((1, _D)),
                _rspec(0), _rspec(1)]
    out_specs = [rows, acc, rows, rows]
    out_shape = [outn, out1, outn, outn]
  else:
    cn, wn1a, wn1b, wn2, bn2, wd1, bd1, wd2p, bd2p = weights

    def body(v_ref, pp_ref, cn_ref, wn1a_ref, wn1b_ref, wn2_ref,
             bn2_ref, wd1_ref, bd1_ref, wd2_ref, bd2_ref, o_ref, sv_ref):
      agg = pp_ref[0] + pp_ref[1]
      h = _lrelu(_dot(v_ref[...], wn1a_ref[...]) + _dot(agg, wn1b_ref[...])
                 + cn_ref[...])
      vn = v_ref[...] + _dot(h, wn2_ref[...]) + bn2_ref[...]

      @pl.when(pl.program_id(0) == 0)
      def _():
        sv_ref[...] = jnp.zeros_like(sv_ref)

      sv_ref[...] += jnp.sum(vn, axis=0, keepdims=True)
      h2 = _lrelu(_dot(vn, wd1_ref[...]) + bd1_ref[...])
      o_ref[...] = _dot(h2, wd2_ref[...]) + bd2_ref[...]

    in_specs = [rows, pp, _wspec((1, _D)), _rspec(0),
                _rspec(1), _wspec((_D, _D)), _wspec((1, _D)),
                _wspec((_D, _D)), _wspec((1, _D)), _wspec((_D, _D)),
                _wspec((1, _D))]
    out_specs = [rows, acc]
    out_shape = [outn, out1]

  return pl.pallas_call(
      body,
      grid=(_N // _BN,),
      in_specs=in_specs,
      out_specs=out_specs,
      out_shape=out_shape,
  )(v, aggp, *weights)


def _g_init_call(u, weg, beg, ew1_full, b1e, nw1_full, bn1):
  def body(u_ref, weg_ref, beg_ref, ew1_ref, b1e_ref, nw1_ref, bn1_ref,
           g_ref, ce_ref, cn_ref):
    g = _lrelu(_dot(u_ref[...], weg_ref[...]) + beg_ref[...])
    g_ref[...] = g
    ce_ref[...] = _dot(g, ew1_ref[...]) + b1e_ref[...]
    cn_ref[...] = _dot(g, nw1_ref[...]) + bn1_ref[...]

  out1 = jax.ShapeDtypeStruct((1, _D), jnp.float32)
  specs = [_wspec((1, 8)), _wspec((8, _D)), _wspec((1, _D)),
           _rspec(3), _wspec((1, _D)), _rspec(2), _wspec((1, _D))]
  return pl.pallas_call(body, grid=(1,), in_specs=specs,
                        out_specs=[_wspec((1, _D))] * 3,
                        out_shape=[out1, out1, out1])(
      u, weg, beg, ew1_full, b1e, nw1_full, bn1)


def _g_update_call(g, sv, se, gw, nxt):
  gw1_full, bg1, wg2, bg2 = gw
  out1 = jax.ShapeDtypeStruct((1, _D), jnp.float32)
  gspecs = [_wspec((1, _D)), _wspec((1, _D)), _wspec((_S, _D)),
            _rspec(0), _rspec(1), _rspec(2), _wspec((1, _D)),
            _wspec((_D, _D)), _wspec((1, _D))]

  if nxt is None:
    def body(g_ref, sv_ref, se_ref, a_ref, b_ref, c_ref, bg1_ref, w2_ref,
             bg2_ref, gout_ref):
      mv = sv_ref[...] * (1.0 / _N)
      me = jnp.sum(se_ref[...], axis=0, keepdims=True) * (1.0 / _E)
      h = _lrelu(_dot(g_ref[...], a_ref[...]) + _dot(mv, b_ref[...])
                 + _dot(me, c_ref[...]) + bg1_ref[...])
      gout_ref[...] = g_ref[...] + _dot(h, w2_ref[...]) + bg2_ref[...]

    return pl.pallas_call(body, grid=(1,), in_specs=gspecs,
                          out_specs=[_wspec((1, _D))], out_shape=[out1])(
        g, sv, se, gw1_full, gw1_full, gw1_full, bg1, wg2, bg2)[0]

  ew1_n, b1e_n, nw1_n, bn1_n = nxt

  def body(g_ref, sv_ref, se_ref, a_ref, b_ref, c_ref, bg1_ref, w2_ref,
           bg2_ref, w1d_ref, b1e_ref, wn1c_ref, bn1_ref,
           gout_ref, ce_ref, cn_ref):
    mv = sv_ref[...] * (1.0 / _N)
    me = jnp.sum(se_ref[...], axis=0, keepdims=True) * (1.0 / _E)
    h = _lrelu(_dot(g_ref[...], a_ref[...]) + _dot(mv, b_ref[...])
               + _dot(me, c_ref[...]) + bg1_ref[...])
    gn = g_ref[...] + _dot(h, w2_ref[...]) + bg2_ref[...]
    gout_ref[...] = gn
    ce_ref[...] = _dot(gn, w1d_ref[...]) + b1e_ref[...]
    cn_ref[...] = _dot(gn, wn1c_ref[...]) + bn1_ref[...]

  specs = gspecs + [_rspec(3), _wspec((1, _D)), _rspec(2), _wspec((1, _D))]
  return pl.pallas_call(body, grid=(1,), in_specs=specs,
                        out_specs=[_wspec((1, _D))] * 3,
                        out_shape=[out1, out1, out1])(
      g, sv, se, gw1_full, gw1_full, gw1_full, bg1, wg2, bg2,
      ew1_n, b1e_n, nw1_n, bn1_n)


def _row(b):
  return b.reshape(1, -1)


def kernel(x, edge_attr, u, edge_index, params):
  p = params
  src = edge_index[0]
  dst = edge_index[1]
  ea_t = edge_attr.T
  wee = p["enc_edge"]["W"]
  bee = _row(p["enc_edge"]["b"])

  blk = p["blocks"]
  ew1 = [b["edge"]["l1"]["W"] for b in blk]
  b1e = [_row(b["edge"]["l1"]["b"]) for b in blk]
  ew2 = [b["edge"]["l2"]["W"] for b in blk]
  b2e = [_row(b["edge"]["l2"]["b"]) for b in blk]

  nw1 = [b["node"]["l1"]["W"] for b in blk]
  bn1 = [_row(b["node"]["l1"]["b"]) for b in blk]
  nw2 = [b["node"]["l2"]["W"] for b in blk]
  bn2 = [_row(b["node"]["l2"]["b"]) for b in blk]

  gws = [(b["glob"]["l1"]["W"], _row(b["glob"]["l1"]["b"]),
          b["glob"]["l2"]["W"], _row(b["glob"]["l2"]["b"])) for b in blk]

  wd1 = p["dec1"]["W"]
  bd1 = _row(p["dec1"]["b"])
  wd2p = jnp.pad(p["dec2"]["W"], ((0, 0), (0, _D - 3)))
  bd2p = jnp.pad(_row(p["dec2"]["b"]), ((0, 0), (0, _D - 3)))

  zeros_init = jnp.zeros((_RPT, _D), jnp.float32)

  v0, a1, b1 = _enc_node_call(x, p["enc_node"]["W"], _row(p["enc_node"]["b"]),
                              ew1[0], ew1[0])
  g0, ce1, cn1 = _g_init_call(u, p["enc_glob"]["W"], _row(p["enc_glob"]["b"]),
                              ew1[0], b1e[0], nw1[0], bn1[0])

  e1 = None
  ses = []
  for si in range(_S):
    gab = _sc_gather_add(a1, b1, src, dst, si)
    e1, se = _edge_call(True, si, ea_t, gab, e1,
                        (wee, bee, ew1[0], ce1, ew2[0], b2e[0]))
    ses.append(se)
  se1 = jnp.concatenate(ses, axis=0)
  agg1 = _sc_segsum(e1, dst, zeros_init)
  v1, sv1, a2, b2 = _node_call(False, v0, agg1,
                               (cn1, nw1[0], nw1[0], nw2[0], bn2[0],
                                ew1[1], ew1[1]))
  g1, ce2, cn2 = _g_update_call(g0, sv1, se1, gws[0],
                                (ew1[1], b1e[1], nw1[1], bn1[1]))

  e2 = None
  ses = []
  for si in range(_S):
    gab = _sc_gather_add(a2, b2, src, dst, si)
    e2, se = _edge_call(False, si, e1, gab, e2,
                        (ew1[1], ce2, ew2[1], b2e[1]))
    ses.append(se)
  se2 = jnp.concatenate(ses, axis=0)
  agg2 = _sc_segsum(e2, dst, zeros_init)
  opad, sv2 = _node_call(True, v1, agg2,
                         (cn2, nw1[1], nw1[1], nw2[1], bn2[1],
                          wd1, bd1, wd2p, bd2p))
  g2 = _g_update_call(g1, sv2, se2, gws[1], None)

  node_out = opad[:, :3]
  return (node_out, e2, g2)
